# baseline (speedup 1.0000x reference)
import functools

import jax
import jax.numpy as jnp
from jax import lax
from jax.experimental import pallas as pl
from jax.experimental.pallas import tpu as pltpu
from jax.experimental.pallas import tpu_sc as plsc

KEEP = 0.1
L = 16
SC_ROWS = 32


def _au(v):
    return lax.bitcast_convert_type(v, jnp.int32) & jnp.int32(0x7FFFFFFF)


def _hist_clear(hist, nbins):
    zeros = jnp.zeros((L,), jnp.int32)

    @plsc.parallel_loop(0, nbins // L, unroll=4)
    def _(j):
        hist[pl.ds(j * L, L)] = zeros


def _hist_pass(xv, hist, n, shift, bmask, prefix_shift, prefix):
    ones = jnp.full((L,), 1, jnp.int32)

    @plsc.parallel_loop(0, n // L, unroll=8)
    def _(i):
        au = _au(xv[pl.ds(i * L, L)])
        b = (au >> shift) & jnp.int32(bmask)
        if prefix_shift >= 32:
            m = jnp.full((L,), True, jnp.bool_)
        else:
            m = (au >> prefix_shift) == prefix
        plsc.addupdate_scatter(hist, [b], ones, mask=m)


def _hist_select(hist, nbins, r):
    iota = lax.iota(jnp.int32, L)
    init = (jnp.int32(0), jnp.int32(0), jnp.int32(0))

    @plsc.parallel_loop(0, nbins // L, unroll=2, carry=init)
    def carry_out(j, carry):
        cum_in, b_sel, above_sel = carry
        start = nbins - (j + 1) * L
        h = hist[pl.ds(start, L)]
        hr = lax.rev(h, (0,))
        cum = jax.lax.cumsum(hr, axis=0) + cum_in
        prev = cum - hr
        is_b = jnp.logical_and(cum >= r, prev < r)
        binv = jnp.int32(nbins - 1) - (jnp.int32(j * L) + iota)
        b_sel = b_sel + jnp.sum(jnp.where(is_b, binv, 0))
        above_sel = above_sel + jnp.sum(jnp.where(is_b, prev, 0))
        cum_out = cum_in + jnp.sum(h)
        return cum_out, b_sel, above_sel

    _, b_sel, above_sel = carry_out
    return b_sel, above_sel


def _make_sc_kernel(B, N, k, rows_per_w):
    mesh = plsc.VectorSubcoreMesh(core_axis_name="c", subcore_axis_name="s")

    @functools.partial(
        pl.kernel,
        mesh=mesh,
        out_type=jax.ShapeDtypeStruct((B, N), jnp.float32),
        scratch_types=[
            pltpu.VMEM((N,), jnp.float32),
            pltpu.VMEM((2048,), jnp.int32),
        ],
        compiler_params=pltpu.CompilerParams(needs_layout_passes=False),
    )
    def sc_k(x_hbm, out_hbm, xv, hist):
        nc = 2
        wid = lax.axis_index("s") * nc + lax.axis_index("c")

        for ri in range(rows_per_w):
            row = wid * rows_per_w + ri
            pltpu.sync_copy(x_hbm.at[row], xv)

            r = jnp.int32(k)
            _hist_clear(hist, 1024)
            _hist_pass(xv, hist, N, 21, 0x3FF, 32, 0)
            b1, above = _hist_select(hist, 1024, r)
            r = r - above
            _hist_clear(hist, 2048)
            _hist_pass(xv, hist, N, 10, 0x7FF, 21, b1)
            b2, above = _hist_select(hist, 2048, r)
            r = r - above
            p12 = (b1 << 11) | b2
            _hist_clear(hist, 1024)
            _hist_pass(xv, hist, N, 0, 0x3FF, 10, p12)
            b3, _ = _hist_select(hist, 1024, r)

            t = (p12 << 10) | b3

            @plsc.parallel_loop(0, N // L, unroll=8)
            def _(i):
                v = xv[pl.ds(i * L, L)]
                xv[pl.ds(i * L, L)] = jnp.where(_au(v) >= t, v, jnp.float32(0.0))

            pltpu.sync_copy(xv, out_hbm.at[row])

    return sc_k


def _tc_body(k, x_ref, o_ref):
    x = x_ref[...]
    u = _au(x)
    Br = x.shape[0]

    def step(_, lohi):
        lo, hi = lohi
        mid = lo + ((hi - lo + jnp.int32(1)) >> 1)
        cnt = jnp.sum((u >= mid).astype(jnp.int32), axis=1, keepdims=True)
        ge = cnt >= k
        return jnp.where(ge, mid, lo), jnp.where(ge, hi, mid - 1)

    lo0 = jnp.zeros((Br, 1), jnp.int32)
    hi0 = jnp.full((Br, 1), 0x7F800000, jnp.int32)
    lo, _ = jax.lax.fori_loop(0, 31, step, (lo0, hi0))
    o_ref[...] = jnp.where(u >= lo, x, jnp.float32(0.0))


def kernel(x):
    B, N = x.shape
    k = max(1, int(N * KEEP))
    sc_rows = SC_ROWS
    tc_rows = B - sc_rows
    sc_out = _make_sc_kernel(sc_rows, N, k, sc_rows // 32)(x[:sc_rows])
    tc_out = pl.pallas_call(
        functools.partial(_tc_body, k),
        out_shape=jax.ShapeDtypeStruct((tc_rows, N), x.dtype),
        in_specs=[pl.BlockSpec(memory_space=pltpu.VMEM)],
        out_specs=pl.BlockSpec(memory_space=pltpu.VMEM),
    )(x[sc_rows:])
    return jnp.concatenate([sc_out, tc_out], axis=0)


# speedup vs baseline: 1.2154x; 1.2154x over previous
import functools

import jax
import jax.numpy as jnp
from jax import lax
from jax.experimental import pallas as pl
from jax.experimental.pallas import tpu as pltpu
from jax.experimental.pallas import tpu_sc as plsc

KEEP = 0.1
L = 16


def _au(v):
    return lax.bitcast_convert_type(v, jnp.int32) & jnp.int32(0x7FFFFFFF)


def _hist_clear(hist, nbins):
    zeros = jnp.zeros((L,), jnp.int32)

    @plsc.parallel_loop(0, nbins // L, unroll=4)
    def _(j):
        hist[pl.ds(j * L, L)] = zeros


def _hist_pass(xv, hist, n, shift, bmask, prefix_shift, prefix):
    ones = jnp.full((L,), 1, jnp.int32)

    @plsc.parallel_loop(0, n // L, unroll=8)
    def _(i):
        au = _au(xv[pl.ds(i * L, L)])
        b = (au >> shift) & jnp.int32(bmask)
        if prefix_shift >= 32:
            m = jnp.full((L,), True, jnp.bool_)
        else:
            m = (au >> prefix_shift) == prefix
        plsc.addupdate_scatter(hist, [b], ones, mask=m)


def _hist_select(hist, nbins, r):
    iota = lax.iota(jnp.int32, L)
    init = (jnp.int32(0), jnp.int32(0), jnp.int32(0))

    @plsc.parallel_loop(0, nbins // L, unroll=2, carry=init)
    def carry_out(j, carry):
        cum_in, b_sel, above_sel = carry
        start = nbins - (j + 1) * L
        h = hist[pl.ds(start, L)]
        hr = lax.rev(h, (0,))
        cum = jax.lax.cumsum(hr, axis=0) + cum_in
        prev = cum - hr
        is_b = jnp.logical_and(cum >= r, prev < r)
        binv = jnp.int32(nbins - 1) - (jnp.int32(j * L) + iota)
        b_sel = b_sel + jnp.sum(jnp.where(is_b, binv, 0))
        above_sel = above_sel + jnp.sum(jnp.where(is_b, prev, 0))
        cum_out = cum_in + jnp.sum(h)
        return cum_out, b_sel, above_sel

    _, b_sel, above_sel = carry_out
    return b_sel, above_sel


def _select_and_mask(xv, hist, n, k):
    r = jnp.int32(k)
    _hist_clear(hist, 1024)
    _hist_pass(xv, hist, n, 21, 0x3FF, 32, 0)
    b1, above = _hist_select(hist, 1024, r)
    r = r - above
    _hist_clear(hist, 2048)
    _hist_pass(xv, hist, n, 10, 0x7FF, 21, b1)
    b2, above = _hist_select(hist, 2048, r)
    r = r - above
    p12 = (b1 << 11) | b2
    _hist_clear(hist, 1024)
    _hist_pass(xv, hist, n, 0, 0x3FF, 10, p12)
    b3, _ = _hist_select(hist, 1024, r)
    t = (p12 << 10) | b3

    @plsc.parallel_loop(0, n // L, unroll=8)
    def _(i):
        v = xv[pl.ds(i * L, L)]
        xv[pl.ds(i * L, L)] = jnp.where(_au(v) >= t, v, jnp.float32(0.0))


def _make_sc_kernel(B, N, k, rows_per_w):
    mesh = plsc.VectorSubcoreMesh(core_axis_name="c", subcore_axis_name="s")

    @functools.partial(
        pl.kernel,
        mesh=mesh,
        out_type=jax.ShapeDtypeStruct((B, N), jnp.float32),
        scratch_types=[
            pltpu.VMEM((N,), jnp.float32),
            pltpu.VMEM((N,), jnp.float32),
            pltpu.VMEM((2048,), jnp.int32),
            pltpu.SemaphoreType.DMA,
            pltpu.SemaphoreType.DMA,
            pltpu.SemaphoreType.DMA,
            pltpu.SemaphoreType.DMA,
        ],
        compiler_params=pltpu.CompilerParams(needs_layout_passes=False),
    )
    def sc_k(x_hbm, out_hbm, xv0, xv1, hist, si0, si1, so0, so1):
        nc = 2
        wid = lax.axis_index("s") * nc + lax.axis_index("c")
        r0 = wid * rows_per_w
        r1 = r0 + 1

        in0 = pltpu.async_copy(x_hbm.at[r0], xv0, si0)
        in1 = pltpu.async_copy(x_hbm.at[r1], xv1, si1)
        in0.wait()
        _select_and_mask(xv0, hist, N, k)
        out0 = pltpu.async_copy(xv0, out_hbm.at[r0], so0)
        in1.wait()
        _select_and_mask(xv1, hist, N, k)
        out1 = pltpu.async_copy(xv1, out_hbm.at[r1], so1)
        out0.wait()
        out1.wait()

    return sc_k


def kernel(x):
    B, N = x.shape
    k = max(1, int(N * KEEP))
    return _make_sc_kernel(B, N, k, B // 32)(x)


# speedup vs baseline: 1.2432x; 1.0229x over previous
import functools

import jax
import jax.numpy as jnp
from jax import lax
from jax.experimental import pallas as pl
from jax.experimental.pallas import tpu as pltpu
from jax.experimental.pallas import tpu_sc as plsc

KEEP = 0.1
L = 16


def _au(v):
    return lax.bitcast_convert_type(v, jnp.int32) & jnp.int32(0x7FFFFFFF)


def _hist_clear(hist, nbins):
    zeros = jnp.zeros((L,), jnp.int32)

    @plsc.parallel_loop(0, nbins // L, unroll=4)
    def _(j):
        hist[pl.ds(j * L, L)] = zeros


def _hist_pass(xv, hist, lo, hi, shift, bmask, prefix_shift, prefix):
    ones = jnp.full((L,), 1, jnp.int32)

    @plsc.parallel_loop(lo // L, hi // L, unroll=8)
    def _(i):
        au = _au(xv[pl.ds(i * L, L)])
        b = (au >> shift) & jnp.int32(bmask)
        if prefix_shift >= 32:
            m = jnp.full((L,), True, jnp.bool_)
        else:
            m = (au >> prefix_shift) == prefix
        plsc.addupdate_scatter(hist, [b], ones, mask=m)


def _hist_select(hist, nbins, r):
    iota = lax.iota(jnp.int32, L)
    zeros = jnp.zeros((L,), jnp.int32)
    init = (jnp.int32(0), jnp.int32(0), jnp.int32(0))

    @plsc.parallel_loop(0, nbins // L, unroll=2, carry=init)
    def carry_out(j, carry):
        cum_in, b_sel, above_sel = carry
        start = nbins - (j + 1) * L
        h = hist[pl.ds(start, L)]
        hist[pl.ds(start, L)] = zeros
        hr = lax.rev(h, (0,))
        cum = jax.lax.cumsum(hr, axis=0) + cum_in
        prev = cum - hr
        is_b = jnp.logical_and(cum >= r, prev < r)
        binv = jnp.int32(nbins - 1) - (jnp.int32(j * L) + iota)
        b_sel = b_sel + jnp.sum(jnp.where(is_b, binv, 0))
        above_sel = above_sel + jnp.sum(jnp.where(is_b, prev, 0))
        cum_out = cum_in + jnp.sum(h)
        return cum_out, b_sel, above_sel

    _, b_sel, above_sel = carry_out
    return b_sel, above_sel


def _select_threshold(xv, hist, n, k):
    r = jnp.int32(k)
    b1, above = _hist_select(hist, 1024, r)
    r = r - above
    _hist_pass(xv, hist, 0, n, 10, 0x7FF, 21, b1)
    b2, above = _hist_select(hist, 2048, r)
    r = r - above
    p12 = (b1 << 11) | b2
    _hist_pass(xv, hist, 0, n, 0, 0x3FF, 10, p12)
    b3, _ = _hist_select(hist, 1024, r)
    return (p12 << 10) | b3


def _mask_pass(xv, t, lo, hi):
    @plsc.parallel_loop(lo // L, hi // L, unroll=8)
    def _(i):
        v = xv[pl.ds(i * L, L)]
        xv[pl.ds(i * L, L)] = jnp.where(_au(v) >= t, v, jnp.float32(0.0))


def _make_sc_kernel(B, N, k, rows_per_w):
    mesh = plsc.VectorSubcoreMesh(core_axis_name="c", subcore_axis_name="s")
    H = N // 2

    @functools.partial(
        pl.kernel,
        mesh=mesh,
        out_type=jax.ShapeDtypeStruct((B, N), jnp.float32),
        scratch_types=[
            pltpu.VMEM((N,), jnp.float32),
            pltpu.VMEM((N,), jnp.float32),
            pltpu.VMEM((2048,), jnp.int32),
            pltpu.SemaphoreType.DMA,
            pltpu.SemaphoreType.DMA,
            pltpu.SemaphoreType.DMA,
            pltpu.SemaphoreType.DMA,
            pltpu.SemaphoreType.DMA,
        ],
        compiler_params=pltpu.CompilerParams(needs_layout_passes=False),
    )
    def sc_k(x_hbm, out_hbm, xv0, xv1, hist, sa, sb, si1, so0, so1):
        nc = 2
        wid = lax.axis_index("s") * nc + lax.axis_index("c")
        r0 = wid * rows_per_w
        r1 = r0 + 1

        inA = pltpu.async_copy(x_hbm.at[r0, pl.ds(0, H)], xv0.at[pl.ds(0, H)], sa)
        inB = pltpu.async_copy(x_hbm.at[r0, pl.ds(H, H)], xv0.at[pl.ds(H, H)], sb)
        in1 = pltpu.async_copy(x_hbm.at[r1], xv1, si1)

        _hist_clear(hist, 2048)
        inA.wait()
        _hist_pass(xv0, hist, 0, H, 21, 0x3FF, 32, 0)
        inB.wait()
        _hist_pass(xv0, hist, H, N, 21, 0x3FF, 32, 0)
        t0 = _select_threshold(xv0, hist, N, k)
        _mask_pass(xv0, t0, 0, N)
        out0 = pltpu.async_copy(xv0, out_hbm.at[r0], so0)

        in1.wait()
        _hist_pass(xv1, hist, 0, N, 21, 0x3FF, 32, 0)
        t1 = _select_threshold(xv1, hist, N, k)
        _mask_pass(xv1, t1, 0, H)
        outA = pltpu.async_copy(xv1.at[pl.ds(0, H)], out_hbm.at[r1, pl.ds(0, H)], so1)
        _mask_pass(xv1, t1, H, N)
        outB = pltpu.async_copy(xv1.at[pl.ds(H, H)], out_hbm.at[r1, pl.ds(H, H)], sa)

        out0.wait()
        outA.wait()
        outB.wait()

    return sc_k


def kernel(x):
    B, N = x.shape
    k = max(1, int(N * KEEP))
    return _make_sc_kernel(B, N, k, B // 32)(x)
